# speedup vs baseline: 4.2727x; 4.2727x over previous
import functools

import jax
import jax.numpy as jnp
from jax import lax
from jax.experimental import pallas as pl
from jax.experimental.pallas import tpu as pltpu
from jax.experimental.pallas import tpu_sc as plsc

B, L, H = 1024, 200, 128
NW = 32
TOK = B * L
TPW = TOK // NW
CHUNK = 128
NCHUNK = TPW // CHUNK

_MESH = plsc.VectorSubcoreMesh(core_axis_name="c", subcore_axis_name="s")


@functools.partial(
    pl.kernel,
    out_type=jax.ShapeDtypeStruct((TOK, H), jnp.float32),
    mesh=_MESH,
    scratch_types=[
        pltpu.VMEM((CHUNK,), jnp.int32),
        pltpu.VMEM((CHUNK,), jnp.int32),
        pltpu.VMEM((CHUNK, H), jnp.float32),
        pltpu.VMEM((CHUNK, H), jnp.float32),
        pltpu.SemaphoreType.DMA,
        pltpu.SemaphoreType.DMA,
    ],
)
def _sc_embed(ids_hbm, tt_hbm, w_hbm, pt_hbm, out_hbm,
              idw_v, idp_v, wbuf, ptbuf, sem_w, sem_p):
    c = lax.axis_index("c")
    s = lax.axis_index("s")
    wid = s * 2 + c
    base0 = wid * TPW
    iota = lax.iota(jnp.int32, 16)

    def chunk_body(g, carry):
        base = base0 + g * CHUNK
        pltpu.sync_copy(ids_hbm.at[pl.ds(base, CHUNK)], idw_v)
        pltpu.sync_copy(tt_hbm.at[pl.ds(base, CHUNK)], idp_v)
        for i in range(CHUNK // 16):
            sl = pl.ds(i * 16, 16)
            pos = lax.rem(base + i * 16 + iota, L)
            idp_v[sl] = pos * 2 + idp_v[sl]
        cp_w = pltpu.async_copy(w_hbm.at[idw_v], wbuf, sem_w)
        cp_p = pltpu.async_copy(pt_hbm.at[idp_v], ptbuf, sem_p)
        cp_w.wait()
        cp_p.wait()

        def row_body(r, carry2):
            for j in range(H // 16):
                sl = pl.ds(j * 16, 16)
                v = wbuf[r, sl] + ptbuf[r, sl]
                wbuf[r, sl] = jnp.minimum(jnp.maximum(v, -1.0), 1.0)
            return carry2

        lax.fori_loop(0, CHUNK, row_body, 0, unroll=2)
        pltpu.sync_copy(wbuf, out_hbm.at[pl.ds(base, CHUNK)])
        return carry

    lax.fori_loop(0, NCHUNK, chunk_body, 0)


def kernel(input_ids, attention_mask, token_type_ids, word_embeddings,
           position_embeddings, token_type_embeddings):
    del attention_mask
    ids = input_ids.reshape(TOK).astype(jnp.int32)
    tt = token_type_ids.reshape(TOK).astype(jnp.int32)
    pt = (position_embeddings[:L, None, :] + token_type_embeddings[None, :, :]
          ).reshape(L * 2, H)
    out = _sc_embed(ids, tt, word_embeddings, pt)
    return out.reshape(B, L, H)


# speedup vs baseline: 6.0110x; 1.4068x over previous
import functools

import jax
import jax.numpy as jnp
from jax import lax
from jax.experimental import pallas as pl
from jax.experimental.pallas import tpu as pltpu
from jax.experimental.pallas import tpu_sc as plsc

B, L, H = 1024, 200, 128
NW = 32
TOK = B * L
TPW = TOK // NW
CHUNK = 128
NCHUNK = TPW // CHUNK

_MESH = plsc.VectorSubcoreMesh(core_axis_name="c", subcore_axis_name="s")

_F32 = jnp.float32
_I32 = jnp.int32


@functools.partial(
    pl.kernel,
    out_type=jax.ShapeDtypeStruct((TOK, H), _F32),
    mesh=_MESH,
    scratch_types=[
        pltpu.VMEM((TPW,), _I32),
        pltpu.VMEM((TPW,), _I32),
        [pltpu.VMEM((CHUNK, H), _F32)] * 2,
        [pltpu.VMEM((CHUNK, H), _F32)] * 2,
        [pltpu.VMEM((CHUNK, H), _F32)] * 2,
        [pltpu.SemaphoreType.DMA] * 2,
        [pltpu.SemaphoreType.DMA] * 2,
        [pltpu.SemaphoreType.DMA] * 2,
    ],
)
def _sc_embed(ids_hbm, tt_hbm, w_hbm, pt_hbm, out_hbm,
              idw, idp, wbufs, ptbufs, obufs, sem_w, sem_p, sem_o):
    c = lax.axis_index("c")
    s = lax.axis_index("s")
    wid = s * 2 + c
    base0 = wid * TPW
    iota = lax.iota(_I32, 16)

    pltpu.sync_copy(ids_hbm.at[pl.ds(base0, TPW)], idw)
    pltpu.sync_copy(tt_hbm.at[pl.ds(base0, TPW)], idp)

    def idx_body(i, carry):
        sl = pl.ds(i * 16, 16)
        pos = lax.rem(i * 16 + iota, L)
        idp[sl] = pos * 2 + idp[sl]
        return carry

    lax.fori_loop(0, TPW // 16, idx_body, 0, unroll=4)

    def launch(g, b):
        sl = pl.ds(g * CHUNK, CHUNK)
        pltpu.async_copy(w_hbm.at[idw.at[sl]], wbufs[b], sem_w[b])
        pltpu.async_copy(pt_hbm.at[idp.at[sl]], ptbufs[b], sem_p[b])

    def wait_gathers(b):
        dummy = w_hbm.at[pl.ds(0, CHUNK)]
        pltpu.make_async_copy(dummy, wbufs[b], sem_w[b]).wait()
        pltpu.make_async_copy(dummy, ptbufs[b], sem_p[b]).wait()

    def wait_out(b):
        pltpu.make_async_copy(obufs[b], out_hbm.at[pl.ds(0, CHUNK)],
                              sem_o[b]).wait()

    def compute(b):
        wb, pb, ob = wbufs[b], ptbufs[b], obufs[b]

        def row_body(r, carry):
            for j in range(H // 16):
                sl = pl.ds(j * 16, 16)
                v = wb[r, sl] + pb[r, sl]
                ob[r, sl] = jnp.minimum(jnp.maximum(v, -1.0), 1.0)
            return carry

        lax.fori_loop(0, CHUNK, row_body, 0, unroll=2)

    def store(g, b):
        pltpu.async_copy(obufs[b], out_hbm.at[pl.ds(base0 + g * CHUNK, CHUNK)],
                         sem_o[b])

    def step(g, b, *, first, last):
        wait_gathers(b)
        if not first:
            wait_out(b)
        compute(b)
        store(g, b)
        if not last:
            launch(g + 2, b)

    launch(0, 0)
    launch(1, 1)
    step(0, 0, first=True, last=False)
    step(1, 1, first=True, last=False)

    def pair_body(go, carry):
        step(2 * go, 0, first=False, last=False)
        step(2 * go + 1, 1, first=False, last=False)
        return carry

    lax.fori_loop(1, NCHUNK // 2 - 1, pair_body, 0)

    step(NCHUNK - 2, 0, first=False, last=True)
    step(NCHUNK - 1, 1, first=False, last=True)
    wait_out(0)
    wait_out(1)


def kernel(input_ids, attention_mask, token_type_ids, word_embeddings,
           position_embeddings, token_type_embeddings):
    del attention_mask
    ids = input_ids.reshape(TOK).astype(_I32)
    tt = token_type_ids.reshape(TOK).astype(_I32)
    pt = (position_embeddings[:L, None, :] + token_type_embeddings[None, :, :]
          ).reshape(L * 2, H)
    out = _sc_embed(ids, tt, word_embeddings, pt)
    return out.reshape(B, L, H)


# speedup vs baseline: 10.9239x; 1.8173x over previous
import functools

import jax
import jax.numpy as jnp
from jax import lax
from jax.experimental import pallas as pl
from jax.experimental.pallas import tpu as pltpu
from jax.experimental.pallas import tpu_sc as plsc

B, L, H = 1024, 200, 128
NW = 32
TOK = B * L
TPW = TOK // NW
CHUNK = 128
NCHUNK = TPW // CHUNK

_MESH = plsc.VectorSubcoreMesh(core_axis_name="c", subcore_axis_name="s")

_F32 = jnp.float32
_I32 = jnp.int32


@functools.partial(
    pl.kernel,
    out_type=jax.ShapeDtypeStruct((TOK, H), _F32),
    mesh=_MESH,
    scratch_types=[
        pltpu.VMEM((TPW,), _I32),
        pltpu.VMEM((TPW,), _I32),
        [pltpu.VMEM((CHUNK, H), _F32)] * 2,
        [pltpu.VMEM((CHUNK, H), _F32)] * 2,
        [pltpu.VMEM((CHUNK, H), _F32)] * 2,
        [pltpu.SemaphoreType.DMA] * 2,
        [pltpu.SemaphoreType.DMA] * 2,
        [pltpu.SemaphoreType.DMA] * 2,
    ],
)
def _sc_embed(ids_hbm, tt_hbm, w_hbm, pt_hbm, out_hbm,
              idw, idp, wbufs, ptbufs, obufs, sem_w, sem_p, sem_o):
    c = lax.axis_index("c")
    s = lax.axis_index("s")
    wid = s * 2 + c
    base0 = wid * TPW
    iota = lax.iota(_I32, 16)

    pltpu.sync_copy(ids_hbm.at[pl.ds(base0, TPW)], idw)
    pltpu.sync_copy(tt_hbm.at[pl.ds(base0, TPW)], idp)

    @plsc.parallel_loop(0, TPW // 16, unroll=4)
    def idx_body(i):
        sl = pl.ds(i * 16, 16)
        pos = lax.rem(i * 16 + iota, L)
        idp[sl] = pos * 2 + idp[sl]

    def launch(g, b):
        sl = pl.ds(g * CHUNK, CHUNK)
        pltpu.async_copy(w_hbm.at[idw.at[sl]], wbufs[b], sem_w[b])
        pltpu.async_copy(pt_hbm.at[idp.at[sl]], ptbufs[b], sem_p[b])

    def wait_gathers(b):
        dummy = w_hbm.at[pl.ds(0, CHUNK)]
        pltpu.make_async_copy(dummy, wbufs[b], sem_w[b]).wait()
        pltpu.make_async_copy(dummy, ptbufs[b], sem_p[b]).wait()

    def wait_out(b):
        pltpu.make_async_copy(obufs[b], out_hbm.at[pl.ds(0, CHUNK)],
                              sem_o[b]).wait()

    def compute(b):
        wb, pb, ob = wbufs[b], ptbufs[b], obufs[b]
        sls = [pl.ds(j * 16, 16) for j in range(H // 16)]

        @plsc.parallel_loop(0, CHUNK, unroll=2)
        def row_body(r):
            ws = [wb[r, sl] for sl in sls]
            ps = [pb[r, sl] for sl in sls]
            for sl, w, p in zip(sls, ws, ps):
                ob[r, sl] = jnp.minimum(jnp.maximum(w + p, -1.0), 1.0)

    def store(g, b):
        pltpu.async_copy(obufs[b], out_hbm.at[pl.ds(base0 + g * CHUNK, CHUNK)],
                         sem_o[b])

    def step(g, b, *, first, last):
        wait_gathers(b)
        if not first:
            wait_out(b)
        compute(b)
        store(g, b)
        if not last:
            launch(g + 2, b)

    launch(0, 0)
    launch(1, 1)
    step(0, 0, first=True, last=False)
    step(1, 1, first=True, last=False)

    def pair_body(go, carry):
        step(2 * go, 0, first=False, last=False)
        step(2 * go + 1, 1, first=False, last=False)
        return carry

    lax.fori_loop(1, NCHUNK // 2 - 1, pair_body, 0)

    step(NCHUNK - 2, 0, first=False, last=True)
    step(NCHUNK - 1, 1, first=False, last=True)
    wait_out(0)
    wait_out(1)


def kernel(input_ids, attention_mask, token_type_ids, word_embeddings,
           position_embeddings, token_type_embeddings):
    del attention_mask
    ids = input_ids.reshape(TOK).astype(_I32)
    tt = token_type_ids.reshape(TOK).astype(_I32)
    pt = (position_embeddings[:L, None, :] + token_type_embeddings[None, :, :]
          ).reshape(L * 2, H)
    out = _sc_embed(ids, tt, word_embeddings, pt)
    return out.reshape(B, L, H)
